# baseline (speedup 1.0000x reference)
import jax
import jax.numpy as jnp
from jax import lax
from jax.experimental import pallas as pl
from jax.experimental.pallas import tpu as pltpu
from jax.experimental.pallas import tpu_sc as plsc

HIDDEN = 32
BATCH = 16384
SUBROWS = 8

_INFO = plsc.get_sparse_core_info()
NC = _INFO.num_cores
NS = _INFO.num_subcores
LANES = _INFO.num_lanes
NW = NC * NS
B_PER_W = BATCH // NW
CHUNK = 16
NCHUNK = B_PER_W // CHUNK


def _mf_body(uid_hbm, iid_hbm, ut_hbm, it_hbm, out_hbm,
             uids_v, iids_v, ublk_v, iblk_v, out_v, sem):
  wid = lax.axis_index("s") * NC + lax.axis_index("c")
  base = wid * B_PER_W
  nblk = ut_hbm.shape[0] // SUBROWS
  ut3 = ut_hbm.reshape(nblk, SUBROWS, HIDDEN)
  it3 = it_hbm.reshape(nblk, SUBROWS, HIDDEN)

  pltpu.sync_copy(uid_hbm.at[pl.ds(base, B_PER_W)], uids_v)
  pltpu.sync_copy(iid_hbm.at[pl.ds(base, B_PER_W)], iids_v)

  def chunk_step(c, carry):
    uidv = uids_v[pl.ds(c * CHUNK, CHUNK)]
    iidv = iids_v[pl.ds(c * CHUNK, CHUNK)]
    ublk = lax.shift_right_logical(uidv, 3)
    iblk = lax.shift_right_logical(iidv, 3)
    for k in range(CHUNK):
      pltpu.async_copy(ut3.at[pl.ds(ublk[k], 1)],
                       ublk_v.at[pl.ds(k, 1)], sem)
      pltpu.async_copy(it3.at[pl.ds(iblk[k], 1)],
                       iblk_v.at[pl.ds(k, 1)], sem)
    for _ in range(2 * CHUNK):
      pltpu.make_async_copy(ut3.at[pl.ds(0, 1)],
                            ublk_v.at[pl.ds(0, 1)], sem).wait()
    usub = lax.bitwise_and(uidv, 7)
    isub = lax.bitwise_and(iidv, 7)
    slots = lax.iota(jnp.int32, LANES)
    acc = jnp.zeros((LANES,), jnp.float32)
    for h in range(HIDDEN):
      hcol = jnp.full((LANES,), h, jnp.int32)
      uc = plsc.load_gather(ublk_v, [slots, usub, hcol])
      ic = plsc.load_gather(iblk_v, [slots, isub, hcol])
      acc = acc + uc * ic
    out_v[pl.ds(c * CHUNK, CHUNK)] = acc
    return carry

  lax.fori_loop(0, NCHUNK, chunk_step, 0)

  pltpu.sync_copy(out_v, out_hbm.at[pl.ds(base, B_PER_W)])


@jax.jit
def _mf(user_ids, item_ids, user_table, item_table):
  mesh = plsc.VectorSubcoreMesh(core_axis_name="c", subcore_axis_name="s")
  kern = pl.kernel(
      _mf_body,
      mesh=mesh,
      out_type=jax.ShapeDtypeStruct((BATCH,), jnp.float32),
      scratch_types=[
          pltpu.VMEM((B_PER_W,), jnp.int32),
          pltpu.VMEM((B_PER_W,), jnp.int32),
          pltpu.VMEM((CHUNK, SUBROWS, HIDDEN), jnp.float32),
          pltpu.VMEM((CHUNK, SUBROWS, HIDDEN), jnp.float32),
          pltpu.VMEM((B_PER_W,), jnp.float32),
          pltpu.SemaphoreType.DMA,
      ],
      compiler_params=pltpu.CompilerParams(needs_layout_passes=False),
  )
  return kern(user_ids, item_ids, user_table, item_table)


def kernel(user_ids, item_ids, user_table, item_table):
  user_ids = user_ids.astype(jnp.int32)
  item_ids = item_ids.astype(jnp.int32)
  return _mf(user_ids, item_ids, user_table, item_table)


# speedup vs baseline: 1.0010x; 1.0010x over previous
import jax
import jax.numpy as jnp
from jax import lax
from jax.experimental import pallas as pl
from jax.experimental.pallas import tpu as pltpu
from jax.experimental.pallas import tpu_sc as plsc

HIDDEN = 32
BATCH = 16384
SUBROWS = 8

_INFO = plsc.get_sparse_core_info()
NC = _INFO.num_cores
NS = _INFO.num_subcores
LANES = _INFO.num_lanes
NW = NC * NS
B_PER_W = BATCH // NW
CHUNK = 16
NCHUNK = B_PER_W // CHUNK


def _mf_body(uid_hbm, iid_hbm, ut_hbm, it_hbm, out_hbm,
             uids_v, iids_v, ublk_v, iblk_v, out_v, sem):
  wid = lax.axis_index("s") * NC + lax.axis_index("c")
  base = wid * B_PER_W

  pltpu.sync_copy(uid_hbm.at[pl.ds(base, B_PER_W)], uids_v)
  pltpu.sync_copy(iid_hbm.at[pl.ds(base, B_PER_W)], iids_v)

  def chunk_step(c, carry):
    uidv = uids_v[pl.ds(c * CHUNK, CHUNK)]
    iidv = iids_v[pl.ds(c * CHUNK, CHUNK)]
    ubase = lax.bitwise_and(uidv, ~7)
    ibase = lax.bitwise_and(iidv, ~7)
    for k in range(CHUNK):
      ustart = pl.multiple_of(ubase[k], SUBROWS)
      istart = pl.multiple_of(ibase[k], SUBROWS)
      pltpu.async_copy(ut_hbm.at[pl.ds(ustart, SUBROWS)],
                       ublk_v.at[pl.ds(k * SUBROWS, SUBROWS)], sem)
      pltpu.async_copy(it_hbm.at[pl.ds(istart, SUBROWS)],
                       iblk_v.at[pl.ds(k * SUBROWS, SUBROWS)], sem)
    for _ in range(2 * CHUNK):
      pltpu.make_async_copy(ut_hbm.at[pl.ds(0, SUBROWS)],
                            ublk_v.at[pl.ds(0, SUBROWS)], sem).wait()
    usub = lax.bitwise_and(uidv, 7)
    isub = lax.bitwise_and(iidv, 7)
    slots = lax.iota(jnp.int32, LANES) * SUBROWS
    urows = slots + usub
    irows = slots + isub
    acc = jnp.zeros((LANES,), jnp.float32)
    for h in range(HIDDEN):
      hcol = jnp.full((LANES,), h, jnp.int32)
      uc = plsc.load_gather(ublk_v, [urows, hcol])
      ic = plsc.load_gather(iblk_v, [irows, hcol])
      acc = acc + uc * ic
    out_v[pl.ds(c * CHUNK, CHUNK)] = acc
    return carry

  lax.fori_loop(0, NCHUNK, chunk_step, 0)

  pltpu.sync_copy(out_v, out_hbm.at[pl.ds(base, B_PER_W)])


@jax.jit
def _mf(user_ids, item_ids, user_table, item_table):
  mesh = plsc.VectorSubcoreMesh(core_axis_name="c", subcore_axis_name="s")
  kern = pl.kernel(
      _mf_body,
      mesh=mesh,
      out_type=jax.ShapeDtypeStruct((BATCH,), jnp.float32),
      scratch_types=[
          pltpu.VMEM((B_PER_W,), jnp.int32),
          pltpu.VMEM((B_PER_W,), jnp.int32),
          pltpu.VMEM((CHUNK * SUBROWS, HIDDEN), jnp.float32),
          pltpu.VMEM((CHUNK * SUBROWS, HIDDEN), jnp.float32),
          pltpu.VMEM((B_PER_W,), jnp.float32),
          pltpu.SemaphoreType.DMA,
      ],
      compiler_params=pltpu.CompilerParams(needs_layout_passes=False),
  )
  return kern(user_ids, item_ids, user_table, item_table)


def kernel(user_ids, item_ids, user_table, item_table):
  user_ids = user_ids.astype(jnp.int32)
  item_ids = item_ids.astype(jnp.int32)
  return _mf(user_ids, item_ids, user_table, item_table)


# speedup vs baseline: 1.0851x; 1.0840x over previous
import jax
import jax.numpy as jnp
from jax import lax
from jax.experimental import pallas as pl
from jax.experimental.pallas import tpu as pltpu
from jax.experimental.pallas import tpu_sc as plsc

HIDDEN = 32
BATCH = 16384
NSEM = 8

_INFO = plsc.get_sparse_core_info()
NC = _INFO.num_cores
NS = _INFO.num_subcores
LANES = _INFO.num_lanes
NW = NC * NS
B_PER_W = BATCH // NW
PASS_IDS = 128
NPASS = B_PER_W // PASS_IDS
GRP_PER_PASS = PASS_IDS // LANES


def _mf_body(uid_hbm, iid_hbm, ut_hbm, it_hbm, out_hbm,
             uids_v, iids_v, urows_v, irows_v, out_v, *sems):
  wid = lax.axis_index("s") * NC + lax.axis_index("c")
  base = wid * B_PER_W

  pltpu.sync_copy(uid_hbm.at[pl.ds(base, B_PER_W)], uids_v)
  pltpu.sync_copy(iid_hbm.at[pl.ds(base, B_PER_W)], iids_v)

  def fire_group(p, g):
    uidv = uids_v[pl.ds(p * PASS_IDS + g * LANES, LANES)]
    iidv = iids_v[pl.ds(p * PASS_IDS + g * LANES, LANES)]
    for k in range(LANES):
      slot = g * LANES + k
      pltpu.async_copy(ut_hbm.at[pl.ds(uidv[k], 1)],
                       urows_v.at[pl.ds(slot, 1)], sems[k % NSEM])
      pltpu.async_copy(it_hbm.at[pl.ds(iidv[k], 1)],
                       irows_v.at[pl.ds(slot, 1)], sems[(k + NSEM // 2) % NSEM])

  def compute_group(p, g):
    rows = g * LANES + lax.iota(jnp.int32, LANES)
    acc = jnp.zeros((LANES,), jnp.float32)
    for h in range(HIDDEN):
      hcol = jnp.full((LANES,), h, jnp.int32)
      uc = plsc.load_gather(urows_v, [rows, hcol])
      ic = plsc.load_gather(irows_v, [rows, hcol])
      acc = acc + uc * ic
    out_v[pl.ds(p * PASS_IDS + g * LANES, LANES)] = acc

  for p in range(NPASS):
    def fstep(g, carry, p=p):
      fire_group(p, g)
      return carry

    lax.fori_loop(0, GRP_PER_PASS, fstep, 0)

    def wstep(_, carry):
      for j in range(NSEM):
        pltpu.make_async_copy(ut_hbm.at[pl.ds(0, 1)],
                              urows_v.at[pl.ds(0, 1)], sems[j]).wait()
      return carry

    lax.fori_loop(0, GRP_PER_PASS * 2 * LANES // NSEM, wstep, 0)

    def cstep(g, carry, p=p):
      compute_group(p, g)
      return carry

    lax.fori_loop(0, GRP_PER_PASS, cstep, 0)

  pltpu.sync_copy(out_v, out_hbm.at[pl.ds(base, B_PER_W)])


@jax.jit
def _mf(user_ids, item_ids, user_table, item_table):
  mesh = plsc.VectorSubcoreMesh(core_axis_name="c", subcore_axis_name="s")
  kern = pl.kernel(
      _mf_body,
      mesh=mesh,
      out_type=jax.ShapeDtypeStruct((BATCH,), jnp.float32),
      scratch_types=[
          pltpu.VMEM((B_PER_W,), jnp.int32),
          pltpu.VMEM((B_PER_W,), jnp.int32),
          pltpu.VMEM((PASS_IDS, HIDDEN), jnp.float32),
          pltpu.VMEM((PASS_IDS, HIDDEN), jnp.float32),
          pltpu.VMEM((B_PER_W,), jnp.float32),
      ] + [pltpu.SemaphoreType.DMA] * NSEM,
      compiler_params=pltpu.CompilerParams(needs_layout_passes=False),
  )
  return kern(user_ids, item_ids, user_table, item_table)


def kernel(user_ids, item_ids, user_table, item_table):
  user_ids = user_ids.astype(jnp.int32)
  item_ids = item_ids.astype(jnp.int32)
  return _mf(user_ids, item_ids, user_table, item_table)
